# baseline (speedup 1.0000x reference)
import functools

import jax
import jax.numpy as jnp
from jax import lax
from jax.experimental import pallas as pl
from jax.experimental.pallas import tpu as pltpu
from jax.experimental.pallas import tpu_sc as plsc

NC = 2
NS = 16
NW = NC * NS
CHUNK = 128


def _make_mesh():
    return plsc.VectorSubcoreMesh(core_axis_name="c", subcore_axis_name="s")


def _make_deg_kernel(nch, span, n_pad):
    @functools.partial(
        pl.kernel,
        out_type=jax.ShapeDtypeStruct((NC, NS, span), jnp.float32),
        mesh=_make_mesh(),
        scratch_types=[
            pltpu.VMEM((nch, CHUNK), jnp.int32),
            pltpu.VMEM((CHUNK,), jnp.float32),
            pltpu.VMEM((span,), jnp.float32),
            pltpu.VMEM_SHARED((n_pad,), jnp.float32),
            pltpu.SemaphoreType.DMA,
        ],
    )
    def deg_kernel(row_hbm, deg_hbm, idx_v, ones_v, zero_v, deg_sh, sem0):
        cid = lax.axis_index("c")
        sid = lax.axis_index("s")
        wid = sid * NC + cid
        pltpu.sync_copy(row_hbm.at[wid], idx_v)

        def fill_ones(j, carry):
            ones_v[pl.ds(j * 16, 16)] = jnp.full((16,), 1.0, jnp.float32)
            return carry

        lax.fori_loop(0, CHUNK // 16, fill_ones, 0)

        def fill_zero(j, carry):
            zero_v[pl.ds(j * 16, 16)] = jnp.zeros((16,), jnp.float32)
            return carry

        lax.fori_loop(0, span // 16, fill_zero, 0)
        pltpu.sync_copy(zero_v, deg_sh.at[pl.ds(sid * span, span)])
        plsc.subcore_barrier()

        def body(j, carry):
            pltpu.async_copy(ones_v, deg_sh.at[idx_v.at[j]], sem0, add=True)
            return carry

        lax.fori_loop(0, nch, body, 0)

        def drain(j, carry):
            pltpu.make_async_copy(ones_v, deg_sh.at[idx_v.at[j]], sem0).wait()
            return carry

        lax.fori_loop(0, nch, drain, 0)
        plsc.subcore_barrier()
        pltpu.sync_copy(deg_sh.at[pl.ds(sid * span, span)], deg_hbm.at[cid, sid])

    return deg_kernel


def _make_edge_kernel(nch, span, n_pad, f):
    @functools.partial(
        pl.kernel,
        out_type=jax.ShapeDtypeStruct((NC, NS, span, f), jnp.float32),
        mesh=_make_mesh(),
        scratch_types=[
            pltpu.VMEM((nch, CHUNK), jnp.int32),
            pltpu.VMEM((2, CHUNK), jnp.int32),
            pltpu.VMEM((2, CHUNK, f), jnp.float32),
            pltpu.VMEM_SHARED((n_pad, f), jnp.float32),
            pltpu.SemaphoreType.DMA,
            pltpu.SemaphoreType.DMA,
            pltpu.SemaphoreType.DMA,
            pltpu.SemaphoreType.DMA,
        ],
    )
    def edge_kernel(y_hbm, col_hbm, row_hbm, out_hbm,
                    col_v, row_v, buf, tmp_sh, semg0, semg1, semr0, semr1):
        cid = lax.axis_index("c")
        sid = lax.axis_index("s")
        wid = sid * NC + cid
        pltpu.sync_copy(col_hbm.at[wid], col_v)

        def fill_zero(t, carry):
            buf[0, t // (f // 16), pl.ds((t % (f // 16)) * 16, 16)] = (
                jnp.zeros((16,), jnp.float32))
            return carry

        lax.fori_loop(0, CHUNK * (f // 16), fill_zero, 0)
        for k in range(span // CHUNK):
            pltpu.sync_copy(
                buf.at[0], tmp_sh.at[pl.ds(sid * span + k * CHUNK, CHUNK)])
        plsc.subcore_barrier()

        pltpu.async_copy(row_hbm.at[wid, 0], row_v.at[0], semr0)
        pltpu.async_copy(y_hbm.at[col_v.at[0]], buf.at[0], semg0)

        def body(g, carry):
            base = g * 2
            pltpu.async_copy(row_hbm.at[wid, base + 1], row_v.at[1], semr1)
            pltpu.async_copy(y_hbm.at[col_v.at[base + 1]], buf.at[1], semg1)
            pltpu.make_async_copy(row_hbm.at[wid, base], row_v.at[0], semr0).wait()
            pltpu.make_async_copy(y_hbm.at[col_v.at[base]], buf.at[0], semg0).wait()
            pltpu.sync_copy(buf.at[0], tmp_sh.at[row_v.at[0]], add=True)

            @pl.when(base + 2 < nch)
            def _():
                pltpu.async_copy(row_hbm.at[wid, base + 2], row_v.at[0], semr0)
                pltpu.async_copy(y_hbm.at[col_v.at[base + 2]], buf.at[0], semg0)

            pltpu.make_async_copy(row_hbm.at[wid, base + 1], row_v.at[1], semr1).wait()
            pltpu.make_async_copy(y_hbm.at[col_v.at[base + 1]], buf.at[1], semg1).wait()
            pltpu.sync_copy(buf.at[1], tmp_sh.at[row_v.at[1]], add=True)
            return carry

        lax.fori_loop(0, nch // 2, body, 0)
        plsc.subcore_barrier()
        pltpu.sync_copy(tmp_sh.at[pl.ds(sid * span, span)], out_hbm.at[cid, sid])

    return edge_kernel


def _dense_body(x_ref, w_ref, v_ref, dp_ref, y_ref, skip_ref):
    d = dp_ref[0] + dp_ref[1]
    dinv = jnp.where(d > 0, lax.rsqrt(d), 0.0)
    xw = jnp.dot(x_ref[...], w_ref[...], preferred_element_type=jnp.float32)
    y_ref[...] = xw * dinv
    skip_ref[...] = jnp.dot(x_ref[...], v_ref[...], preferred_element_type=jnp.float32)


def _final_body(t_ref, dp_ref, skip_ref, b_ref, o_ref):
    d = dp_ref[0] + dp_ref[1]
    dinv = jnp.where(d > 0, lax.rsqrt(d), 0.0)
    agg = -(t_ref[0] + t_ref[1]) * dinv
    o_ref[...] = jnp.maximum(agg + skip_ref[...] + b_ref[...], 0.0)


def kernel(x, edge_index, W, V, B):
    n, f = x.shape
    e = edge_index.shape[1]

    per_dma = NW * CHUNK
    nch = -(-e // per_dma)
    nch += nch % 2
    e_pad = nch * per_dma
    span = -(-(n + 1) // NS)
    span = -(-span // CHUNK) * CHUNK
    n_pad = NS * span

    row = edge_index[0]
    col = edge_index[1]
    pad_rows = n + jnp.arange(e_pad - e, dtype=jnp.int32) % (n_pad - n)
    row_t = jnp.concatenate([row, pad_rows]).reshape(NW, nch, CHUNK)
    col_t = jnp.concatenate(
        [col, jnp.zeros((e_pad - e,), dtype=jnp.int32)]).reshape(NW, nch, CHUNK)

    deg_p = _make_deg_kernel(nch, span, n_pad)(row_t)
    deg_p3 = deg_p.reshape(NC, n_pad, 1)

    bn = 2000
    grid = (n // bn,)
    y, skip = pl.pallas_call(
        _dense_body,
        grid=grid,
        in_specs=[
            pl.BlockSpec((bn, f), lambda i: (i, 0)),
            pl.BlockSpec((f, f), lambda i: (0, 0)),
            pl.BlockSpec((f, f), lambda i: (0, 0)),
            pl.BlockSpec((NC, bn, 1), lambda i: (0, i, 0)),
        ],
        out_specs=[pl.BlockSpec((bn, f), lambda i: (i, 0))] * 2,
        out_shape=[jax.ShapeDtypeStruct((n, f), jnp.float32)] * 2,
    )(x, W[0], V[0], deg_p3)

    tmp = _make_edge_kernel(nch, span, n_pad, f)(y, col_t, row_t)
    tmp = tmp.reshape(NC, n_pad, f)

    out = pl.pallas_call(
        _final_body,
        grid=grid,
        in_specs=[
            pl.BlockSpec((NC, bn, f), lambda i: (0, i, 0)),
            pl.BlockSpec((NC, bn, 1), lambda i: (0, i, 0)),
            pl.BlockSpec((bn, f), lambda i: (i, 0)),
            pl.BlockSpec((1, f), lambda i: (0, 0)),
        ],
        out_specs=pl.BlockSpec((bn, f), lambda i: (i, 0)),
        out_shape=jax.ShapeDtypeStruct((n, f), jnp.float32),
    )(tmp, deg_p3, skip, B[0])
    return out


# speedup vs baseline: 1.0124x; 1.0124x over previous
import functools

import jax
import jax.numpy as jnp
from jax import lax
from jax.experimental import pallas as pl
from jax.experimental.pallas import tpu as pltpu
from jax.experimental.pallas import tpu_sc as plsc

NC = 2
NS = 16
NW = NC * NS
CHUNK = 128


def _make_mesh():
    return plsc.VectorSubcoreMesh(core_axis_name="c", subcore_axis_name="s")


def _make_deg_kernel(nch, span, n_pad):
    @functools.partial(
        pl.kernel,
        out_type=jax.ShapeDtypeStruct((NC, NS, span), jnp.float32),
        mesh=_make_mesh(),
        scratch_types=[
            pltpu.VMEM((nch, CHUNK), jnp.int32),
            pltpu.VMEM((CHUNK,), jnp.float32),
            pltpu.VMEM((span,), jnp.float32),
            pltpu.VMEM_SHARED((n_pad,), jnp.float32),
            pltpu.SemaphoreType.DMA,
        ],
    )
    def deg_kernel(row_hbm, deg_hbm, idx_v, ones_v, zero_v, deg_sh, sem0):
        cid = lax.axis_index("c")
        sid = lax.axis_index("s")
        wid = sid * NC + cid
        pltpu.sync_copy(row_hbm.at[wid], idx_v)

        def fill_ones(j, carry):
            ones_v[pl.ds(j * 16, 16)] = jnp.full((16,), 1.0, jnp.float32)
            return carry

        lax.fori_loop(0, CHUNK // 16, fill_ones, 0)

        def fill_zero(j, carry):
            zero_v[pl.ds(j * 16, 16)] = jnp.zeros((16,), jnp.float32)
            return carry

        lax.fori_loop(0, span // 16, fill_zero, 0)
        pltpu.sync_copy(zero_v, deg_sh.at[pl.ds(sid * span, span)])
        plsc.subcore_barrier()

        def body(j, carry):
            pltpu.async_copy(ones_v, deg_sh.at[idx_v.at[j]], sem0, add=True)
            return carry

        lax.fori_loop(0, nch, body, 0)

        def drain(j, carry):
            pltpu.make_async_copy(ones_v, deg_sh.at[idx_v.at[j]], sem0).wait()
            return carry

        lax.fori_loop(0, nch, drain, 0)
        plsc.subcore_barrier()
        pltpu.sync_copy(deg_sh.at[pl.ds(sid * span, span)], deg_hbm.at[cid, sid])

    return deg_kernel


def _make_edge_kernel(nch, span, n_pad, f):
    @functools.partial(
        pl.kernel,
        out_type=jax.ShapeDtypeStruct((NC, NS, span, f), jnp.float32),
        mesh=_make_mesh(),
        scratch_types=[
            pltpu.VMEM((nch, CHUNK), jnp.int32),
            pltpu.VMEM((2, CHUNK), jnp.int32),
            pltpu.VMEM((2, CHUNK, f), jnp.float32),
            pltpu.VMEM_SHARED((n_pad, f), jnp.float32),
            pltpu.SemaphoreType.DMA,
            pltpu.SemaphoreType.DMA,
            pltpu.SemaphoreType.DMA,
            pltpu.SemaphoreType.DMA,
        ],
    )
    def edge_kernel(y_hbm, col_hbm, row_hbm, out_hbm,
                    col_v, row_v, buf, tmp_sh, semg0, semg1, semr0, semr1):
        cid = lax.axis_index("c")
        sid = lax.axis_index("s")
        wid = sid * NC + cid
        pltpu.sync_copy(col_hbm.at[wid], col_v)

        def fill_zero(t, carry):
            buf[0, t // (f // 16), pl.ds((t % (f // 16)) * 16, 16)] = (
                jnp.zeros((16,), jnp.float32))
            return carry

        lax.fori_loop(0, CHUNK * (f // 16), fill_zero, 0)
        for k in range(span // CHUNK):
            pltpu.sync_copy(
                buf.at[0], tmp_sh.at[pl.ds(sid * span + k * CHUNK, CHUNK)])
        plsc.subcore_barrier()

        pltpu.async_copy(row_hbm.at[wid, 0], row_v.at[0], semr0)
        pltpu.async_copy(y_hbm.at[col_v.at[0]], buf.at[0], semg0)

        def body(g, carry):
            base = g * 2
            pltpu.async_copy(row_hbm.at[wid, base + 1], row_v.at[1], semr1)
            pltpu.async_copy(y_hbm.at[col_v.at[base + 1]], buf.at[1], semg1)
            pltpu.make_async_copy(row_hbm.at[wid, base], row_v.at[0], semr0).wait()
            pltpu.make_async_copy(y_hbm.at[col_v.at[base]], buf.at[0], semg0).wait()
            pass

            @pl.when(base + 2 < nch)
            def _():
                pltpu.async_copy(row_hbm.at[wid, base + 2], row_v.at[0], semr0)
                pltpu.async_copy(y_hbm.at[col_v.at[base + 2]], buf.at[0], semg0)

            pltpu.make_async_copy(row_hbm.at[wid, base + 1], row_v.at[1], semr1).wait()
            pltpu.make_async_copy(y_hbm.at[col_v.at[base + 1]], buf.at[1], semg1).wait()
            pass
            return carry

        lax.fori_loop(0, nch // 2, body, 0)
        plsc.subcore_barrier()
        pltpu.sync_copy(tmp_sh.at[pl.ds(sid * span, span)], out_hbm.at[cid, sid])

    return edge_kernel


def _dense_body(x_ref, w_ref, v_ref, dp_ref, y_ref, skip_ref):
    d = dp_ref[0] + dp_ref[1]
    dinv = jnp.where(d > 0, lax.rsqrt(d), 0.0)
    xw = jnp.dot(x_ref[...], w_ref[...], preferred_element_type=jnp.float32)
    y_ref[...] = xw * dinv
    skip_ref[...] = jnp.dot(x_ref[...], v_ref[...], preferred_element_type=jnp.float32)


def _final_body(t_ref, dp_ref, skip_ref, b_ref, o_ref):
    d = dp_ref[0] + dp_ref[1]
    dinv = jnp.where(d > 0, lax.rsqrt(d), 0.0)
    agg = -(t_ref[0] + t_ref[1]) * dinv
    o_ref[...] = jnp.maximum(agg + skip_ref[...] + b_ref[...], 0.0)


def kernel(x, edge_index, W, V, B):
    n, f = x.shape
    e = edge_index.shape[1]

    per_dma = NW * CHUNK
    nch = -(-e // per_dma)
    nch += nch % 2
    e_pad = nch * per_dma
    span = -(-(n + 1) // NS)
    span = -(-span // CHUNK) * CHUNK
    n_pad = NS * span

    row = edge_index[0]
    col = edge_index[1]
    pad_rows = n + jnp.arange(e_pad - e, dtype=jnp.int32) % (n_pad - n)
    row_t = jnp.concatenate([row, pad_rows]).reshape(NW, nch, CHUNK)
    col_t = jnp.concatenate(
        [col, jnp.zeros((e_pad - e,), dtype=jnp.int32)]).reshape(NW, nch, CHUNK)

    deg_p = _make_deg_kernel(nch, span, n_pad)(row_t)
    deg_p3 = deg_p.reshape(NC, n_pad, 1)

    bn = 2000
    grid = (n // bn,)
    y, skip = pl.pallas_call(
        _dense_body,
        grid=grid,
        in_specs=[
            pl.BlockSpec((bn, f), lambda i: (i, 0)),
            pl.BlockSpec((f, f), lambda i: (0, 0)),
            pl.BlockSpec((f, f), lambda i: (0, 0)),
            pl.BlockSpec((NC, bn, 1), lambda i: (0, i, 0)),
        ],
        out_specs=[pl.BlockSpec((bn, f), lambda i: (i, 0))] * 2,
        out_shape=[jax.ShapeDtypeStruct((n, f), jnp.float32)] * 2,
    )(x, W[0], V[0], deg_p3)

    tmp = _make_edge_kernel(nch, span, n_pad, f)(y, col_t, row_t)
    tmp = tmp.reshape(NC, n_pad, f)

    out = pl.pallas_call(
        _final_body,
        grid=grid,
        in_specs=[
            pl.BlockSpec((NC, bn, f), lambda i: (0, i, 0)),
            pl.BlockSpec((NC, bn, 1), lambda i: (0, i, 0)),
            pl.BlockSpec((bn, f), lambda i: (i, 0)),
            pl.BlockSpec((1, f), lambda i: (0, 0)),
        ],
        out_specs=pl.BlockSpec((bn, f), lambda i: (i, 0)),
        out_shape=jax.ShapeDtypeStruct((n, f), jnp.float32),
    )(tmp, deg_p3, skip, B[0])
    return out


# speedup vs baseline: 1.7515x; 1.7301x over previous
import functools

import jax
import jax.numpy as jnp
from jax import lax
from jax.experimental import pallas as pl
from jax.experimental.pallas import tpu as pltpu
from jax.experimental.pallas import tpu_sc as plsc

NC = 2
NS = 16
CHUNK = 128
NCH0 = 130
NCH1 = 28


def _make_mesh():
    return plsc.VectorSubcoreMesh(core_axis_name="c", subcore_axis_name="s")


def _make_deg_kernel(span, n_pad):
    @functools.partial(
        pl.kernel,
        out_type=jax.ShapeDtypeStruct((NC, NS, span), jnp.float32),
        mesh=_make_mesh(),
        scratch_types=[
            pltpu.VMEM((NCH0, CHUNK), jnp.int32),
            pltpu.VMEM((CHUNK,), jnp.float32),
            pltpu.VMEM((span,), jnp.float32),
            pltpu.VMEM_SHARED((n_pad,), jnp.float32),
            pltpu.SemaphoreType.DMA,
        ],
    )
    def deg_kernel(row_hbm, deg_hbm, idx_v, ones_v, zero_v, deg_sh, sem0):
        cid = lax.axis_index("c")
        sid = lax.axis_index("s")
        nch_my = jnp.where(cid == 0, NCH0, NCH1)
        pltpu.sync_copy(row_hbm.at[cid, sid], idx_v)

        def fill_ones(j, carry):
            ones_v[pl.ds(j * 16, 16)] = jnp.full((16,), 1.0, jnp.float32)
            return carry

        lax.fori_loop(0, CHUNK // 16, fill_ones, 0)

        def fill_zero(j, carry):
            zero_v[pl.ds(j * 16, 16)] = jnp.zeros((16,), jnp.float32)
            return carry

        lax.fori_loop(0, span // 16, fill_zero, 0)
        pltpu.sync_copy(zero_v, deg_sh.at[pl.ds(sid * span, span)])
        plsc.subcore_barrier()

        def body(j, carry):
            pltpu.async_copy(ones_v, deg_sh.at[idx_v.at[j]], sem0, add=True)
            return carry

        lax.fori_loop(0, nch_my, body, 0)

        def drain(j, carry):
            pltpu.make_async_copy(ones_v, deg_sh.at[idx_v.at[j]], sem0).wait()
            return carry

        lax.fori_loop(0, nch_my, drain, 0)
        plsc.subcore_barrier()
        pltpu.sync_copy(deg_sh.at[pl.ds(sid * span, span)], deg_hbm.at[cid, sid])

    return deg_kernel


def _make_edge_kernel(span, n_pad, f):
    @functools.partial(
        pl.kernel,
        out_type=jax.ShapeDtypeStruct((NC, NS, span, f), jnp.float32),
        mesh=_make_mesh(),
        scratch_types=[
            pltpu.VMEM((2, CHUNK), jnp.int32),
            pltpu.VMEM((2, CHUNK), jnp.int32),
            pltpu.VMEM((2, CHUNK, f), jnp.float32),
            pltpu.VMEM_SHARED((n_pad, f), jnp.float32),
            pltpu.SemaphoreType.DMA,
            pltpu.SemaphoreType.DMA,
            pltpu.SemaphoreType.DMA,
            pltpu.SemaphoreType.DMA,
            pltpu.SemaphoreType.DMA,
            pltpu.SemaphoreType.DMA,
        ],
    )
    def edge_kernel(y_hbm, col_hbm, row_hbm, out_hbm,
                    cring, rring, buf, tmp_sh,
                    semg0, semg1, semc0, semc1, semr0, semr1):
        cid = lax.axis_index("c")
        sid = lax.axis_index("s")
        nch_my = jnp.where(cid == 0, NCH0, NCH1)

        def fill_zero(t, carry):
            buf[0, t // (f // 16), pl.ds((t % (f // 16)) * 16, 16)] = (
                jnp.zeros((16,), jnp.float32))
            return carry

        lax.fori_loop(0, CHUNK * (f // 16), fill_zero, 0)
        for k in range(span // CHUNK):
            pltpu.sync_copy(
                buf.at[0], tmp_sh.at[pl.ds(sid * span + k * CHUNK, CHUNK)])
        rem = span % CHUNK
        if rem:
            pltpu.sync_copy(
                buf.at[0, pl.ds(0, rem)],
                tmp_sh.at[pl.ds(sid * span + (span // CHUNK) * CHUNK, rem)])
        plsc.subcore_barrier()

        pltpu.async_copy(col_hbm.at[cid, sid, 0], cring.at[0], semc0)
        pltpu.async_copy(col_hbm.at[cid, sid, 1], cring.at[1], semc1)
        pltpu.async_copy(row_hbm.at[cid, sid, 0], rring.at[0], semr0)
        pltpu.async_copy(row_hbm.at[cid, sid, 1], rring.at[1], semr1)
        pltpu.make_async_copy(col_hbm.at[cid, sid, 0], cring.at[0], semc0).wait()
        pltpu.async_copy(y_hbm.at[cring.at[0]], buf.at[0], semg0)

        def chunk_step(j, sc, sr, sg, sc_o, sr_o, sg_o, slot, other):
            @pl.when(j + 1 < nch_my)
            def _():
                pltpu.make_async_copy(
                    col_hbm.at[cid, sid, j + 1], cring.at[other], sc_o).wait()
                pltpu.async_copy(y_hbm.at[cring.at[other]], buf.at[other], sg_o)

            pltpu.make_async_copy(y_hbm.at[cring.at[slot]], buf.at[slot], sg).wait()

            @pl.when(j + 2 < nch_my)
            def _():
                pltpu.async_copy(col_hbm.at[cid, sid, j + 2], cring.at[slot], sc)

            pltpu.make_async_copy(
                row_hbm.at[cid, sid, j], rring.at[slot], sr).wait()
            pltpu.sync_copy(buf.at[slot], tmp_sh.at[rring.at[slot]], add=True)

            @pl.when(j + 2 < nch_my)
            def _():
                pltpu.async_copy(row_hbm.at[cid, sid, j + 2], rring.at[slot], sr)

        def body(g, carry):
            base = g * 2
            chunk_step(base, semc0, semr0, semg0, semc1, semr1, semg1, 0, 1)
            chunk_step(base + 1, semc1, semr1, semg1, semc0, semr0, semg0, 1, 0)
            return carry

        lax.fori_loop(0, nch_my // 2, body, 0)
        plsc.subcore_barrier()
        pltpu.sync_copy(tmp_sh.at[pl.ds(sid * span, span)], out_hbm.at[cid, sid])

    return edge_kernel


def _dense_body(x_ref, w_ref, v_ref, dp_ref, y_ref, skip_ref):
    d = dp_ref[0] + dp_ref[1]
    dinv = jnp.where(d > 0, lax.rsqrt(d), 0.0)
    xw = jnp.dot(x_ref[...], w_ref[...], preferred_element_type=jnp.float32)
    y_ref[...] = xw * dinv
    skip_ref[...] = jnp.dot(x_ref[...], v_ref[...], preferred_element_type=jnp.float32)


def _final_body(t_ref, dp_ref, skip_ref, b_ref, o_ref):
    d = dp_ref[0] + dp_ref[1]
    dinv = jnp.where(d > 0, lax.rsqrt(d), 0.0)
    agg = -(t_ref[0] + t_ref[1]) * dinv
    o_ref[...] = jnp.maximum(agg + skip_ref[...] + b_ref[...], 0.0)


def kernel(x, edge_index, W, V, B):
    n, f = x.shape
    e = edge_index.shape[1]

    e0 = NS * NCH0 * CHUNK
    e_pad = e0 + NS * NCH1 * CHUNK
    span = -(-(n + 1) // NS)
    span += (-span) % 8
    n_pad = NS * span
    span_d = span + (-span) % 16
    n_pad_d = NS * span_d

    row = edge_index[0]
    col = edge_index[1]
    pad_rows = n + jnp.arange(e_pad - e, dtype=jnp.int32) % (n_pad - n)
    row_p = jnp.concatenate([row, pad_rows])
    col_p = jnp.concatenate([col, jnp.zeros((e_pad - e,), dtype=jnp.int32)])

    def to_tiles(a, dummy):
        c0 = a[:e0].reshape(NS, NCH0, CHUNK)
        c1 = a[e0:].reshape(NS, NCH1, CHUNK)
        c1 = jnp.concatenate(
            [c1, jnp.broadcast_to(dummy, (NS, NCH0 - NCH1, CHUNK))], axis=1)
        return jnp.stack([c0, c1])

    row_t = to_tiles(row_p, jnp.int32(n))
    col_t = to_tiles(col_p, jnp.int32(0))

    deg_p = _make_deg_kernel(span_d, n_pad_d)(row_t)
    deg_p3 = deg_p.reshape(NC, n_pad_d, 1)

    bn = 2000
    grid = (n // bn,)
    y, skip = pl.pallas_call(
        _dense_body,
        grid=grid,
        in_specs=[
            pl.BlockSpec((bn, f), lambda i: (i, 0)),
            pl.BlockSpec((f, f), lambda i: (0, 0)),
            pl.BlockSpec((f, f), lambda i: (0, 0)),
            pl.BlockSpec((NC, bn, 1), lambda i: (0, i, 0)),
        ],
        out_specs=[pl.BlockSpec((bn, f), lambda i: (i, 0))] * 2,
        out_shape=[jax.ShapeDtypeStruct((n, f), jnp.float32)] * 2,
    )(x, W[0], V[0], deg_p3)

    tmp = _make_edge_kernel(span, n_pad, f)(y, col_t, row_t)
    tmp = tmp.reshape(NC, n_pad, f)

    out = pl.pallas_call(
        _final_body,
        grid=grid,
        in_specs=[
            pl.BlockSpec((NC, bn, f), lambda i: (0, i, 0)),
            pl.BlockSpec((NC, bn, 1), lambda i: (0, i, 0)),
            pl.BlockSpec((bn, f), lambda i: (i, 0)),
            pl.BlockSpec((1, f), lambda i: (0, 0)),
        ],
        out_specs=pl.BlockSpec((bn, f), lambda i: (i, 0)),
        out_shape=jax.ShapeDtypeStruct((n, f), jnp.float32),
    )(tmp, deg_p3, skip, B[0])
    return out


# speedup vs baseline: 1.7738x; 1.0128x over previous
import functools

import jax
import jax.numpy as jnp
from jax import lax
from jax.experimental import pallas as pl
from jax.experimental.pallas import tpu as pltpu
from jax.experimental.pallas import tpu_sc as plsc

NC = 2
NS = 16
CHUNK = 128
NCH0 = 136
NCH1 = 22


def _make_mesh():
    return plsc.VectorSubcoreMesh(core_axis_name="c", subcore_axis_name="s")


def _make_deg_kernel(span, n_pad):
    @functools.partial(
        pl.kernel,
        out_type=jax.ShapeDtypeStruct((NC, NS, span), jnp.float32),
        mesh=_make_mesh(),
        scratch_types=[
            pltpu.VMEM((NCH0, CHUNK), jnp.int32),
            pltpu.VMEM((CHUNK,), jnp.float32),
            pltpu.VMEM((span,), jnp.float32),
            pltpu.VMEM_SHARED((n_pad,), jnp.float32),
            pltpu.SemaphoreType.DMA,
        ],
    )
    def deg_kernel(row_hbm, deg_hbm, idx_v, ones_v, zero_v, deg_sh, sem0):
        cid = lax.axis_index("c")
        sid = lax.axis_index("s")
        nch_my = jnp.where(cid == 0, NCH0, NCH1)
        pltpu.sync_copy(row_hbm.at[cid, sid], idx_v)

        def fill_ones(j, carry):
            ones_v[pl.ds(j * 16, 16)] = jnp.full((16,), 1.0, jnp.float32)
            return carry

        lax.fori_loop(0, CHUNK // 16, fill_ones, 0)

        def fill_zero(j, carry):
            zero_v[pl.ds(j * 16, 16)] = jnp.zeros((16,), jnp.float32)
            return carry

        lax.fori_loop(0, span // 16, fill_zero, 0)
        pltpu.sync_copy(zero_v, deg_sh.at[pl.ds(sid * span, span)])
        plsc.subcore_barrier()

        window = 64

        def body(j, carry):
            pltpu.async_copy(ones_v, deg_sh.at[idx_v.at[j]], sem0, add=True)

            @pl.when(j >= window)
            def _():
                pltpu.make_async_copy(
                    ones_v, deg_sh.at[idx_v.at[j - window]], sem0).wait()

            return carry

        lax.fori_loop(0, nch_my, body, 0)

        def drain(j, carry):
            pltpu.make_async_copy(ones_v, deg_sh.at[idx_v.at[j]], sem0).wait()
            return carry

        lax.fori_loop(jnp.maximum(nch_my - window, 0), nch_my, drain, 0)
        plsc.subcore_barrier()
        pltpu.sync_copy(deg_sh.at[pl.ds(sid * span, span)], deg_hbm.at[cid, sid])

    return deg_kernel


def _make_edge_kernel(span, n_pad, f):
    @functools.partial(
        pl.kernel,
        out_type=jax.ShapeDtypeStruct((NC, NS, span, f), jnp.float32),
        mesh=_make_mesh(),
        scratch_types=[
            pltpu.VMEM((2, CHUNK), jnp.int32),
            pltpu.VMEM((2, CHUNK), jnp.int32),
            pltpu.VMEM((2, CHUNK, f), jnp.float32),
            pltpu.VMEM_SHARED((n_pad, f), jnp.float32),
            pltpu.SemaphoreType.DMA,
            pltpu.SemaphoreType.DMA,
            pltpu.SemaphoreType.DMA,
            pltpu.SemaphoreType.DMA,
            pltpu.SemaphoreType.DMA,
            pltpu.SemaphoreType.DMA,
        ],
    )
    def edge_kernel(y_hbm, col_hbm, row_hbm, out_hbm,
                    cring, rring, buf, tmp_sh,
                    semg0, semg1, semc0, semc1, semr0, semr1):
        cid = lax.axis_index("c")
        sid = lax.axis_index("s")
        nch_my = jnp.where(cid == 0, NCH0, NCH1)

        def fill_zero(t, carry):
            buf[0, t // (f // 16), pl.ds((t % (f // 16)) * 16, 16)] = (
                jnp.zeros((16,), jnp.float32))
            return carry

        lax.fori_loop(0, CHUNK * (f // 16), fill_zero, 0)
        for k in range(span // CHUNK):
            pltpu.sync_copy(
                buf.at[0], tmp_sh.at[pl.ds(sid * span + k * CHUNK, CHUNK)])
        rem = span % CHUNK
        if rem:
            pltpu.sync_copy(
                buf.at[0, pl.ds(0, rem)],
                tmp_sh.at[pl.ds(sid * span + (span // CHUNK) * CHUNK, rem)])
        plsc.subcore_barrier()

        pltpu.async_copy(col_hbm.at[cid, sid, 0], cring.at[0], semc0)
        pltpu.async_copy(col_hbm.at[cid, sid, 1], cring.at[1], semc1)
        pltpu.async_copy(row_hbm.at[cid, sid, 0], rring.at[0], semr0)
        pltpu.async_copy(row_hbm.at[cid, sid, 1], rring.at[1], semr1)
        pltpu.make_async_copy(col_hbm.at[cid, sid, 0], cring.at[0], semc0).wait()
        pltpu.async_copy(y_hbm.at[cring.at[0]], buf.at[0], semg0)

        def chunk_step(j, sc, sr, sg, sc_o, sr_o, sg_o, slot, other):
            @pl.when(j + 1 < nch_my)
            def _():
                pltpu.make_async_copy(
                    col_hbm.at[cid, sid, j + 1], cring.at[other], sc_o).wait()
                pltpu.async_copy(y_hbm.at[cring.at[other]], buf.at[other], sg_o)

            pltpu.make_async_copy(y_hbm.at[cring.at[slot]], buf.at[slot], sg).wait()

            @pl.when(j + 2 < nch_my)
            def _():
                pltpu.async_copy(col_hbm.at[cid, sid, j + 2], cring.at[slot], sc)

            pltpu.make_async_copy(
                row_hbm.at[cid, sid, j], rring.at[slot], sr).wait()
            pltpu.sync_copy(buf.at[slot], tmp_sh.at[rring.at[slot]], add=True)

            @pl.when(j + 2 < nch_my)
            def _():
                pltpu.async_copy(row_hbm.at[cid, sid, j + 2], rring.at[slot], sr)

        def body(g, carry):
            base = g * 2
            chunk_step(base, semc0, semr0, semg0, semc1, semr1, semg1, 0, 1)
            chunk_step(base + 1, semc1, semr1, semg1, semc0, semr0, semg0, 1, 0)
            return carry

        lax.fori_loop(0, nch_my // 2, body, 0)
        plsc.subcore_barrier()
        pltpu.sync_copy(tmp_sh.at[pl.ds(sid * span, span)], out_hbm.at[cid, sid])

    return edge_kernel


def _dense_body(x_ref, w_ref, v_ref, dp_ref, y_ref, skip_ref):
    d = dp_ref[0] + dp_ref[1]
    dinv = jnp.where(d > 0, lax.rsqrt(d), 0.0)
    xw = jnp.dot(x_ref[...], w_ref[...], preferred_element_type=jnp.float32)
    y_ref[...] = xw * dinv
    skip_ref[...] = jnp.dot(x_ref[...], v_ref[...], preferred_element_type=jnp.float32)


def _final_body(t_ref, dp_ref, skip_ref, b_ref, o_ref):
    d = dp_ref[0] + dp_ref[1]
    dinv = jnp.where(d > 0, lax.rsqrt(d), 0.0)
    agg = -(t_ref[0] + t_ref[1]) * dinv
    o_ref[...] = jnp.maximum(agg + skip_ref[...] + b_ref[...], 0.0)


def kernel(x, edge_index, W, V, B):
    n, f = x.shape
    e = edge_index.shape[1]

    e0 = NS * NCH0 * CHUNK
    e_pad = e0 + NS * NCH1 * CHUNK
    span = -(-(n + 1) // NS)
    span += (-span) % 8
    n_pad = NS * span
    span_d = span + (-span) % 16
    n_pad_d = NS * span_d

    row = edge_index[0]
    col = edge_index[1]
    pad_rows = n + jnp.arange(e_pad - e, dtype=jnp.int32) % (n_pad - n)
    row_p = jnp.concatenate([row, pad_rows])
    col_p = jnp.concatenate([col, jnp.zeros((e_pad - e,), dtype=jnp.int32)])

    def to_tiles(a, dummy):
        c0 = a[:e0].reshape(NS, NCH0, CHUNK)
        c1 = a[e0:].reshape(NS, NCH1, CHUNK)
        c1 = jnp.concatenate(
            [c1, jnp.broadcast_to(dummy, (NS, NCH0 - NCH1, CHUNK))], axis=1)
        return jnp.stack([c0, c1])

    row_t = to_tiles(row_p, jnp.int32(n))
    col_t = to_tiles(col_p, jnp.int32(0))

    deg_p = _make_deg_kernel(span_d, n_pad_d)(row_t)
    deg_p3 = deg_p.reshape(NC, n_pad_d, 1)

    bn = 2000
    grid = (n // bn,)
    y, skip = pl.pallas_call(
        _dense_body,
        grid=grid,
        in_specs=[
            pl.BlockSpec((bn, f), lambda i: (i, 0)),
            pl.BlockSpec((f, f), lambda i: (0, 0)),
            pl.BlockSpec((f, f), lambda i: (0, 0)),
            pl.BlockSpec((NC, bn, 1), lambda i: (0, i, 0)),
        ],
        out_specs=[pl.BlockSpec((bn, f), lambda i: (i, 0))] * 2,
        out_shape=[jax.ShapeDtypeStruct((n, f), jnp.float32)] * 2,
    )(x, W[0], V[0], deg_p3)

    tmp = _make_edge_kernel(span, n_pad, f)(y, col_t, row_t)
    tmp = tmp.reshape(NC, n_pad, f)

    out = pl.pallas_call(
        _final_body,
        grid=grid,
        in_specs=[
            pl.BlockSpec((NC, bn, f), lambda i: (0, i, 0)),
            pl.BlockSpec((NC, bn, 1), lambda i: (0, i, 0)),
            pl.BlockSpec((bn, f), lambda i: (i, 0)),
            pl.BlockSpec((1, f), lambda i: (0, 0)),
        ],
        out_specs=pl.BlockSpec((bn, f), lambda i: (i, 0)),
        out_shape=jax.ShapeDtypeStruct((n, f), jnp.float32),
    )(tmp, deg_p3, skip, B[0])
    return out


# speedup vs baseline: 2.0129x; 1.1348x over previous
import functools

import jax
import jax.numpy as jnp
from jax import lax
from jax.experimental import pallas as pl
from jax.experimental.pallas import tpu as pltpu
from jax.experimental.pallas import tpu_sc as plsc

NC = 2
NS = 16
CHUNK = 112
NCH0 = 156
NCH1 = 24


def _make_mesh():
    return plsc.VectorSubcoreMesh(core_axis_name="c", subcore_axis_name="s")


def _make_deg_kernel(span, n_pad):
    @functools.partial(
        pl.kernel,
        out_type=jax.ShapeDtypeStruct((NC, NS, span), jnp.float32),
        mesh=_make_mesh(),
        scratch_types=[
            pltpu.VMEM((NCH0, CHUNK), jnp.int32),
            pltpu.VMEM((CHUNK,), jnp.float32),
            pltpu.VMEM((span,), jnp.float32),
            pltpu.VMEM_SHARED((n_pad,), jnp.float32),
            pltpu.SemaphoreType.DMA,
        ],
    )
    def deg_kernel(row_hbm, deg_hbm, idx_v, ones_v, zero_v, deg_sh, sem0):
        cid = lax.axis_index("c")
        sid = lax.axis_index("s")
        nch_my = jnp.where(cid == 0, NCH0, NCH1)
        pltpu.sync_copy(row_hbm.at[cid, sid], idx_v)

        def fill_ones(j, carry):
            ones_v[pl.ds(j * 16, 16)] = jnp.full((16,), 1.0, jnp.float32)
            return carry

        lax.fori_loop(0, CHUNK // 16, fill_ones, 0)

        def fill_zero(j, carry):
            zero_v[pl.ds(j * 16, 16)] = jnp.zeros((16,), jnp.float32)
            return carry

        lax.fori_loop(0, span // 16, fill_zero, 0)
        pltpu.sync_copy(zero_v, deg_sh.at[pl.ds(sid * span, span)])
        plsc.subcore_barrier()

        window = 64

        def body(j, carry):
            pltpu.async_copy(ones_v, deg_sh.at[idx_v.at[j]], sem0, add=True)

            @pl.when(j >= window)
            def _():
                pltpu.make_async_copy(
                    ones_v, deg_sh.at[idx_v.at[j - window]], sem0).wait()

            return carry

        lax.fori_loop(0, nch_my, body, 0)

        def drain(j, carry):
            pltpu.make_async_copy(ones_v, deg_sh.at[idx_v.at[j]], sem0).wait()
            return carry

        lax.fori_loop(jnp.maximum(nch_my - window, 0), nch_my, drain, 0)
        plsc.subcore_barrier()
        pltpu.sync_copy(deg_sh.at[pl.ds(sid * span, span)], deg_hbm.at[cid, sid])

    return deg_kernel


def _make_edge_kernel(span, n_pad, f):
    @functools.partial(
        pl.kernel,
        out_type=jax.ShapeDtypeStruct((NC, NS, span, f), jnp.float32),
        mesh=_make_mesh(),
        scratch_types=[
            pltpu.VMEM((3, CHUNK), jnp.int32),
            pltpu.VMEM((3, CHUNK), jnp.int32),
            pltpu.VMEM((3, CHUNK, f), jnp.float32),
            pltpu.VMEM_SHARED((n_pad, f), jnp.float32),
            pltpu.SemaphoreType.DMA,
            pltpu.SemaphoreType.DMA,
            pltpu.SemaphoreType.DMA,
            pltpu.SemaphoreType.DMA,
            pltpu.SemaphoreType.DMA,
            pltpu.SemaphoreType.DMA,
            pltpu.SemaphoreType.DMA,
            pltpu.SemaphoreType.DMA,
            pltpu.SemaphoreType.DMA,
        ],
    )
    def edge_kernel(y_hbm, col_hbm, row_hbm, out_hbm,
                    cring, rring, buf, tmp_sh,
                    semg0, semg1, semg2, semc0, semc1, semc2,
                    semr0, semr1, semr2):
        cid = lax.axis_index("c")
        sid = lax.axis_index("s")
        nch_my = jnp.where(cid == 0, NCH0, NCH1)

        def fill_zero(t, carry):
            buf[0, t // (f // 16), pl.ds((t % (f // 16)) * 16, 16)] = (
                jnp.zeros((16,), jnp.float32))
            return carry

        lax.fori_loop(0, CHUNK * (f // 16), fill_zero, 0)
        for k in range(span // CHUNK):
            pltpu.sync_copy(
                buf.at[0], tmp_sh.at[pl.ds(sid * span + k * CHUNK, CHUNK)])
        rem = span % CHUNK
        if rem:
            pltpu.sync_copy(
                buf.at[0, pl.ds(0, rem)],
                tmp_sh.at[pl.ds(sid * span + (span // CHUNK) * CHUNK, rem)])
        plsc.subcore_barrier()

        semc = (semc0, semc1, semc2)
        semr = (semr0, semr1, semr2)
        semg = (semg0, semg1, semg2)
        for s in range(3):
            pltpu.async_copy(col_hbm.at[cid, sid, s], cring.at[s], semc[s])
            pltpu.async_copy(row_hbm.at[cid, sid, s], rring.at[s], semr[s])
        for s in range(2):
            pltpu.make_async_copy(
                col_hbm.at[cid, sid, s], cring.at[s], semc[s]).wait()
            pltpu.async_copy(y_hbm.at[cring.at[s]], buf.at[s], semg[s])

        def chunk_step(j, slot):
            nxt = (slot + 2) % 3
            @pl.when(j + 2 < nch_my)
            def _():
                pltpu.make_async_copy(
                    col_hbm.at[cid, sid, j + 2], cring.at[nxt], semc[nxt]).wait()
                pltpu.async_copy(y_hbm.at[cring.at[nxt]], buf.at[nxt], semg[nxt])

            pltpu.make_async_copy(
                y_hbm.at[cring.at[slot]], buf.at[slot], semg[slot]).wait()

            @pl.when(j + 3 < nch_my)
            def _():
                pltpu.async_copy(
                    col_hbm.at[cid, sid, j + 3], cring.at[slot], semc[slot])

            pltpu.make_async_copy(
                row_hbm.at[cid, sid, j], rring.at[slot], semr[slot]).wait()
            pltpu.sync_copy(buf.at[slot], tmp_sh.at[rring.at[slot]], add=True)

            @pl.when(j + 3 < nch_my)
            def _():
                pltpu.async_copy(
                    row_hbm.at[cid, sid, j + 3], rring.at[slot], semr[slot])

        def body(g, carry):
            base = g * 3
            chunk_step(base, 0)
            chunk_step(base + 1, 1)
            chunk_step(base + 2, 2)
            return carry

        lax.fori_loop(0, nch_my // 3, body, 0)
        plsc.subcore_barrier()
        pltpu.sync_copy(tmp_sh.at[pl.ds(sid * span, span)], out_hbm.at[cid, sid])

    return edge_kernel


def _dense_body(x_ref, w_ref, v_ref, dp_ref, y_ref, skip_ref):
    d = dp_ref[0] + dp_ref[1]
    dinv = jnp.where(d > 0, lax.rsqrt(d), 0.0)
    xw = jnp.dot(x_ref[...], w_ref[...], preferred_element_type=jnp.float32)
    y_ref[...] = xw * dinv
    skip_ref[...] = jnp.dot(x_ref[...], v_ref[...], preferred_element_type=jnp.float32)


def _final_body(t_ref, dp_ref, skip_ref, b_ref, o_ref):
    d = dp_ref[0] + dp_ref[1]
    dinv = jnp.where(d > 0, lax.rsqrt(d), 0.0)
    agg = -(t_ref[0] + t_ref[1]) * dinv
    o_ref[...] = jnp.maximum(agg + skip_ref[...] + b_ref[...], 0.0)


def kernel(x, edge_index, W, V, B):
    n, f = x.shape
    e = edge_index.shape[1]

    e0 = NS * NCH0 * CHUNK
    e_pad = e0 + NS * NCH1 * CHUNK
    span = -(-(n + 1) // NS)
    span += (-span) % 8
    n_pad = NS * span
    span_d = span + (-span) % 16
    n_pad_d = NS * span_d

    row = edge_index[0]
    col = edge_index[1]
    pad_rows = n + jnp.arange(e_pad - e, dtype=jnp.int32) % (n_pad - n)
    row_p = jnp.concatenate([row, pad_rows])
    col_p = jnp.concatenate([col, jnp.zeros((e_pad - e,), dtype=jnp.int32)])

    def to_tiles(a, dummy):
        c0 = a[:e0].reshape(NS, NCH0, CHUNK)
        c1 = a[e0:].reshape(NS, NCH1, CHUNK)
        c1 = jnp.concatenate(
            [c1, jnp.broadcast_to(dummy, (NS, NCH0 - NCH1, CHUNK))], axis=1)
        return jnp.stack([c0, c1])

    row_t = to_tiles(row_p, jnp.int32(n))
    col_t = to_tiles(col_p, jnp.int32(0))

    deg_p = _make_deg_kernel(span_d, n_pad_d)(row_t)
    deg_p3 = deg_p.reshape(NC, n_pad_d, 1)

    bn = 2000
    grid = (n // bn,)
    y, skip = pl.pallas_call(
        _dense_body,
        grid=grid,
        in_specs=[
            pl.BlockSpec((bn, f), lambda i: (i, 0)),
            pl.BlockSpec((f, f), lambda i: (0, 0)),
            pl.BlockSpec((f, f), lambda i: (0, 0)),
            pl.BlockSpec((NC, bn, 1), lambda i: (0, i, 0)),
        ],
        out_specs=[pl.BlockSpec((bn, f), lambda i: (i, 0))] * 2,
        out_shape=[jax.ShapeDtypeStruct((n, f), jnp.float32)] * 2,
    )(x, W[0], V[0], deg_p3)

    tmp = _make_edge_kernel(span, n_pad, f)(y, col_t, row_t)
    tmp = tmp.reshape(NC, n_pad, f)

    out = pl.pallas_call(
        _final_body,
        grid=grid,
        in_specs=[
            pl.BlockSpec((NC, bn, f), lambda i: (0, i, 0)),
            pl.BlockSpec((NC, bn, 1), lambda i: (0, i, 0)),
            pl.BlockSpec((bn, f), lambda i: (i, 0)),
            pl.BlockSpec((1, f), lambda i: (0, 0)),
        ],
        out_specs=pl.BlockSpec((bn, f), lambda i: (i, 0)),
        out_shape=jax.ShapeDtypeStruct((n, f), jnp.float32),
    )(tmp, deg_p3, skip, B[0])
    return out


# speedup vs baseline: 2.0571x; 1.0220x over previous
import functools

import jax
import jax.numpy as jnp
from jax import lax
from jax.experimental import pallas as pl
from jax.experimental.pallas import tpu as pltpu
from jax.experimental.pallas import tpu_sc as plsc

NC = 2
NS = 16
CHUNK = 112
NCH0 = 159
NCH1 = 21


def _make_mesh():
    return plsc.VectorSubcoreMesh(core_axis_name="c", subcore_axis_name="s")


def _make_deg_kernel(span, n_pad):
    @functools.partial(
        pl.kernel,
        out_type=jax.ShapeDtypeStruct((NC, NS, span), jnp.float32),
        mesh=_make_mesh(),
        scratch_types=[
            pltpu.VMEM((NCH0, CHUNK), jnp.int32),
            pltpu.VMEM((CHUNK,), jnp.float32),
            pltpu.VMEM((span,), jnp.float32),
            pltpu.VMEM_SHARED((n_pad,), jnp.float32),
            pltpu.SemaphoreType.DMA,
        ],
    )
    def deg_kernel(row_hbm, deg_hbm, idx_v, ones_v, zero_v, deg_sh, sem0):
        cid = lax.axis_index("c")
        sid = lax.axis_index("s")
        nch_my = jnp.where(cid == 0, NCH0, NCH1)
        pltpu.sync_copy(row_hbm.at[cid, sid], idx_v)

        def fill_ones(j, carry):
            ones_v[pl.ds(j * 16, 16)] = jnp.full((16,), 1.0, jnp.float32)
            return carry

        lax.fori_loop(0, CHUNK // 16, fill_ones, 0)

        def fill_zero(j, carry):
            zero_v[pl.ds(j * 16, 16)] = jnp.zeros((16,), jnp.float32)
            return carry

        lax.fori_loop(0, span // 16, fill_zero, 0)
        pltpu.sync_copy(zero_v, deg_sh.at[pl.ds(sid * span, span)])
        plsc.subcore_barrier()

        window = 64

        def body(j, carry):
            pltpu.async_copy(ones_v, deg_sh.at[idx_v.at[j]], sem0, add=True)

            @pl.when(j >= window)
            def _():
                pltpu.make_async_copy(
                    ones_v, deg_sh.at[idx_v.at[j - window]], sem0).wait()

            return carry

        lax.fori_loop(0, nch_my, body, 0)

        def drain(j, carry):
            pltpu.make_async_copy(ones_v, deg_sh.at[idx_v.at[j]], sem0).wait()
            return carry

        lax.fori_loop(jnp.maximum(nch_my - window, 0), nch_my, drain, 0)
        plsc.subcore_barrier()
        pltpu.sync_copy(deg_sh.at[pl.ds(sid * span, span)], deg_hbm.at[cid, sid])

    return deg_kernel


def _make_edge_kernel(span, n_pad, f):
    @functools.partial(
        pl.kernel,
        out_type=jax.ShapeDtypeStruct((NC, NS, span, f), jnp.float32),
        mesh=_make_mesh(),
        scratch_types=[
            pltpu.VMEM((3, CHUNK), jnp.int32),
            pltpu.VMEM((3, CHUNK), jnp.int32),
            pltpu.VMEM((3, CHUNK, f), jnp.float32),
            pltpu.VMEM_SHARED((n_pad, f), jnp.float32),
            pltpu.SemaphoreType.DMA,
            pltpu.SemaphoreType.DMA,
            pltpu.SemaphoreType.DMA,
            pltpu.SemaphoreType.DMA,
            pltpu.SemaphoreType.DMA,
            pltpu.SemaphoreType.DMA,
            pltpu.SemaphoreType.DMA,
            pltpu.SemaphoreType.DMA,
            pltpu.SemaphoreType.DMA,
        ],
    )
    def edge_kernel(y_hbm, col_hbm, row_hbm, out_hbm,
                    cring, rring, buf, tmp_sh,
                    semg0, semg1, semg2, semc0, semc1, semc2,
                    semr0, semr1, semr2):
        cid = lax.axis_index("c")
        sid = lax.axis_index("s")
        nch_my = jnp.where(cid == 0, NCH0, NCH1)

        def fill_zero(t, carry):
            buf[0, t // (f // 16), pl.ds((t % (f // 16)) * 16, 16)] = (
                jnp.zeros((16,), jnp.float32))
            return carry

        lax.fori_loop(0, CHUNK * (f // 16), fill_zero, 0)
        for k in range(span // CHUNK):
            pltpu.sync_copy(
                buf.at[0], tmp_sh.at[pl.ds(sid * span + k * CHUNK, CHUNK)])
        rem = span % CHUNK
        if rem:
            pltpu.sync_copy(
                buf.at[0, pl.ds(0, rem)],
                tmp_sh.at[pl.ds(sid * span + (span // CHUNK) * CHUNK, rem)])
        plsc.subcore_barrier()

        semc = (semc0, semc1, semc2)
        semr = (semr0, semr1, semr2)
        semg = (semg0, semg1, semg2)
        for s in range(3):
            pltpu.async_copy(col_hbm.at[cid, sid, s], cring.at[s], semc[s])
            pltpu.async_copy(row_hbm.at[cid, sid, s], rring.at[s], semr[s])
        for s in range(2):
            pltpu.make_async_copy(
                col_hbm.at[cid, sid, s], cring.at[s], semc[s]).wait()
            pltpu.async_copy(y_hbm.at[cring.at[s]], buf.at[s], semg[s])

        def chunk_step(j, slot):
            nxt = (slot + 2) % 3
            @pl.when(j + 2 < nch_my)
            def _():
                pltpu.make_async_copy(
                    col_hbm.at[cid, sid, j + 2], cring.at[nxt], semc[nxt]).wait()
                pltpu.async_copy(y_hbm.at[cring.at[nxt]], buf.at[nxt], semg[nxt])

            pltpu.make_async_copy(
                y_hbm.at[cring.at[slot]], buf.at[slot], semg[slot]).wait()

            @pl.when(j + 3 < nch_my)
            def _():
                pltpu.async_copy(
                    col_hbm.at[cid, sid, j + 3], cring.at[slot], semc[slot])

            pltpu.make_async_copy(
                row_hbm.at[cid, sid, j], rring.at[slot], semr[slot]).wait()
            pltpu.sync_copy(buf.at[slot], tmp_sh.at[rring.at[slot]], add=True)

            @pl.when(j + 3 < nch_my)
            def _():
                pltpu.async_copy(
                    row_hbm.at[cid, sid, j + 3], rring.at[slot], semr[slot])

        def body(g, carry):
            base = g * 3
            chunk_step(base, 0)
            chunk_step(base + 1, 1)
            chunk_step(base + 2, 2)
            return carry

        lax.fori_loop(0, nch_my // 3, body, 0)
        plsc.subcore_barrier()
        pltpu.sync_copy(tmp_sh.at[pl.ds(sid * span, span)], out_hbm.at[cid, sid])

    return edge_kernel


def _dense_body(x_ref, w_ref, v_ref, dp_ref, y_ref, skip_ref):
    d = dp_ref[0] + dp_ref[1]
    dinv = jnp.where(d > 0, lax.rsqrt(d), 0.0)
    xw = jnp.dot(x_ref[...], w_ref[...], preferred_element_type=jnp.float32)
    y_ref[...] = xw * dinv
    skip_ref[...] = jnp.dot(x_ref[...], v_ref[...], preferred_element_type=jnp.float32)


def _final_body(t_ref, dp_ref, skip_ref, b_ref, o_ref):
    d = dp_ref[0] + dp_ref[1]
    dinv = jnp.where(d > 0, lax.rsqrt(d), 0.0)
    agg = -(t_ref[0] + t_ref[1]) * dinv
    o_ref[...] = jnp.maximum(agg + skip_ref[...] + b_ref[...], 0.0)


def kernel(x, edge_index, W, V, B):
    n, f = x.shape
    e = edge_index.shape[1]

    e0 = NS * NCH0 * CHUNK
    e_pad = e0 + NS * NCH1 * CHUNK
    span = -(-(n + 1) // NS)
    span += (-span) % 8
    n_pad = NS * span
    span_d = span + (-span) % 16
    n_pad_d = NS * span_d

    row = edge_index[0]
    col = edge_index[1]
    pad_rows = n + jnp.arange(e_pad - e, dtype=jnp.int32) % (n_pad - n)
    row_p = jnp.concatenate([row, pad_rows])
    col_p = jnp.concatenate([col, jnp.zeros((e_pad - e,), dtype=jnp.int32)])

    def to_tiles(a, dummy):
        c0 = a[:e0].reshape(NS, NCH0, CHUNK)
        c1 = a[e0:].reshape(NS, NCH1, CHUNK)
        c1 = jnp.concatenate(
            [c1, jnp.broadcast_to(dummy, (NS, NCH0 - NCH1, CHUNK))], axis=1)
        return jnp.stack([c0, c1])

    row_t = to_tiles(row_p, jnp.int32(n))
    col_t = to_tiles(col_p, jnp.int32(0))

    deg_p = _make_deg_kernel(span_d, n_pad_d)(row_t)
    deg_p3 = deg_p.reshape(NC, n_pad_d, 1)

    bn = 2000
    grid = (n // bn,)
    y, skip = pl.pallas_call(
        _dense_body,
        grid=grid,
        in_specs=[
            pl.BlockSpec((bn, f), lambda i: (i, 0)),
            pl.BlockSpec((f, f), lambda i: (0, 0)),
            pl.BlockSpec((f, f), lambda i: (0, 0)),
            pl.BlockSpec((NC, bn, 1), lambda i: (0, i, 0)),
        ],
        out_specs=[pl.BlockSpec((bn, f), lambda i: (i, 0))] * 2,
        out_shape=[jax.ShapeDtypeStruct((n, f), jnp.float32)] * 2,
    )(x, W[0], V[0], deg_p3)

    tmp = _make_edge_kernel(span, n_pad, f)(y, col_t, row_t)
    tmp = tmp.reshape(NC, n_pad, f)

    out = pl.pallas_call(
        _final_body,
        grid=grid,
        in_specs=[
            pl.BlockSpec((NC, bn, f), lambda i: (0, i, 0)),
            pl.BlockSpec((NC, bn, 1), lambda i: (0, i, 0)),
            pl.BlockSpec((bn, f), lambda i: (i, 0)),
            pl.BlockSpec((1, f), lambda i: (0, 0)),
        ],
        out_specs=pl.BlockSpec((bn, f), lambda i: (i, 0)),
        out_shape=jax.ShapeDtypeStruct((n, f), jnp.float32),
    )(tmp, deg_p3, skip, B[0])
    return out
